# baseline (speedup 1.0000x reference)
import functools

import jax
import jax.numpy as jnp
from jax.experimental import pallas as pl
from jax.experimental.pallas import tpu as pltpu

ALPHA = 0.2
LOG2E = 1.4426950408889634
NHA = 640

BM = 1024
BMM = 512


def _wh_outputs(i, wh, asd_ref, wh_ref, s_ref, dt_ref, maxd_ref):
    sd = jnp.dot(wh, asd_ref[...], preferred_element_type=jnp.float32)
    s_ref[...] = sd[:, 0:1]
    d = sd[:, 1:2]
    dt_ref[...] = d.T
    wh_ref[:, :512] = wh.astype(jnp.bfloat16)
    lane = jax.lax.broadcasted_iota(jnp.int32, (wh.shape[0], NHA - 512), 1)
    wh_ref[:, 512:] = jnp.where(lane == 0, 1.0, 0.0).astype(jnp.bfloat16)
    local_max = jnp.max(d, axis=0, keepdims=True)

    @pl.when(i == 0)
    def _first():
        maxd_ref[...] = local_max

    @pl.when(i > 0)
    def _rest():
        maxd_ref[...] = jnp.maximum(maxd_ref[...], local_max)


def _mm_kernel(h_ref, w_ref, asd_ref, wh_ref, s_ref, dt_ref, maxd_ref):
    i = pl.program_id(0)
    wh = jnp.dot(h_ref[...], w_ref[...], preferred_element_type=jnp.float32)
    _wh_outputs(i, wh, asd_ref, wh_ref, s_ref, dt_ref, maxd_ref)


def _wh_specs(n, bm):
    out_specs = [
        pl.BlockSpec((bm, NHA), lambda i: (i, 0)),
        pl.BlockSpec((bm, 1), lambda i: (i, 0)),
        pl.BlockSpec((1, bm), lambda i: (0, i)),
        pl.BlockSpec((1, 1), lambda i: (0, 0)),
    ]
    out_shape = [
        jax.ShapeDtypeStruct((n, NHA), jnp.bfloat16),
        jax.ShapeDtypeStruct((n, 1), jnp.float32),
        jax.ShapeDtypeStruct((1, n), jnp.float32),
        jax.ShapeDtypeStruct((1, 1), jnp.float32),
    ]
    return out_specs, out_shape


def _asd(a):
    nh = a.shape[0] // 2
    return jnp.concatenate([a[:nh], a[nh:]], axis=1) * LOG2E


def _mm(h, w, a):
    n, nin = h.shape
    out_specs, out_shape = _wh_specs(n, BMM)
    return pl.pallas_call(
        _mm_kernel,
        grid=(n // BMM,),
        in_specs=[
            pl.BlockSpec((BMM, nin), lambda i: (i, 0)),
            pl.BlockSpec((nin, w.shape[1]), lambda i: (0, 0)),
            pl.BlockSpec((w.shape[1], 2), lambda i: (0, 0)),
        ],
        out_specs=out_specs,
        out_shape=out_shape,
        compiler_params=pltpu.CompilerParams(
            dimension_semantics=("arbitrary",),
        ),
    )(h, w, _asd(a))


KC = 1024


def _attn_out(mask_chunk, wh_ref, s_ref, dt_ref, maxd_ref):
    s = s_ref[...]
    x = s + maxd_ref[...]
    m = jnp.maximum(x, ALPHA * x)
    a = s - m
    b = ALPHA * s - m
    dt = dt_ref[...]
    n = dt.shape[1]
    pv = None
    for c in range(n // KC):
        dtc = dt[:, c * KC:(c + 1) * KC]
        t = jnp.maximum(a + dtc, b + ALPHA * dtc)
        p16 = jnp.exp2(t).astype(jnp.bfloat16) * mask_chunk(c)
        part = jnp.dot(p16, wh_ref[pl.ds(c * KC, KC), :],
                       preferred_element_type=jnp.float32)
        pv = part if pv is None else pv + part
    return pv[:, :512] / pv[:, 512:513]


def _attn1_kernel(adj_ref, wh_ref, s_ref, dt_ref, maxd_ref, wn_ref, asd_ref,
                  mask_ref, who_ref, so_ref, dto_ref, maxdo_ref):
    i = pl.program_id(0)
    masked = adj_ref[...] > 0.99
    mask_ref[...] = masked.astype(jnp.int8)

    def mask_chunk(c):
        return masked[:, c * KC:(c + 1) * KC].astype(jnp.bfloat16)

    h = _attn_out(mask_chunk, wh_ref, s_ref, dt_ref, maxd_ref)
    wh = jnp.dot(jnp.maximum(h, 0.0), wn_ref[...],
                 preferred_element_type=jnp.float32)
    _wh_outputs(i, wh, asd_ref, who_ref, so_ref, dto_ref, maxdo_ref)


def _attn2_kernel(mask_ref, wh_ref, s_ref, dt_ref, maxd_ref, wn_ref, asd_ref,
                  who_ref, so_ref, dto_ref, maxdo_ref):
    i = pl.program_id(0)
    def mask_chunk(c):
        return mask_ref[:, pl.ds(c * KC, KC)].astype(jnp.bfloat16)

    h = _attn_out(mask_chunk, wh_ref, s_ref, dt_ref, maxd_ref)
    wh = jnp.dot(jnp.maximum(h, 0.0), wn_ref[...],
                 preferred_element_type=jnp.float32)
    _wh_outputs(i, wh, asd_ref, who_ref, so_ref, dto_ref, maxdo_ref)


def _attn3_kernel(mask_ref, wh_ref, s_ref, dt_ref, maxd_ref, o_ref):
    def mask_chunk(c):
        return mask_ref[:, pl.ds(c * KC, KC)].astype(jnp.bfloat16)

    o_ref[...] = _attn_out(mask_chunk, wh_ref, s_ref, dt_ref, maxd_ref)


def _attn_specs(n, first):
    mat_dtype = jnp.float32 if first else jnp.bfloat16
    in_specs = [
        pl.BlockSpec((BM, n), lambda i: (i, 0)),
        pl.BlockSpec((n, NHA), lambda i: (0, 0)),
        pl.BlockSpec((BM, 1), lambda i: (i, 0)),
        pl.BlockSpec((1, n), lambda i: (0, 0)),
        pl.BlockSpec((1, 1), lambda i: (0, 0)),
    ]
    return mat_dtype, in_specs


def _attn_mid(mat, wh, s, dt, maxd, w_next, a_next, first):
    n = s.shape[0]
    _, in_specs = _attn_specs(n, first)
    in_specs += [
        pl.BlockSpec((512, 512), lambda i: (0, 0)),
        pl.BlockSpec((512, 2), lambda i: (0, 0)),
    ]
    out_specs, out_shape = _wh_specs(n, BM)
    if first:
        body = _attn1_kernel
        out_specs = [pl.BlockSpec((BM, n), lambda i: (i, 0))] + out_specs
        out_shape = [jax.ShapeDtypeStruct((n, n), jnp.int8)] + out_shape
    else:
        body = _attn2_kernel
    return pl.pallas_call(
        body,
        grid=(n // BM,),
        in_specs=in_specs,
        out_specs=out_specs,
        out_shape=out_shape,
        compiler_params=pltpu.CompilerParams(
            dimension_semantics=("arbitrary",),
        ),
    )(mat, wh, s, dt, maxd, w_next, _asd(a_next))


def _attn_last(mask, wh, s, dt, maxd):
    n = s.shape[0]
    _, in_specs = _attn_specs(n, False)
    return pl.pallas_call(
        _attn3_kernel,
        grid=(n // BM,),
        in_specs=in_specs,
        out_specs=pl.BlockSpec((BM, 512), lambda i: (i, 0)),
        out_shape=jax.ShapeDtypeStruct((n, 512), jnp.float32),
        compiler_params=pltpu.CompilerParams(
            dimension_semantics=("parallel",),
        ),
    )(mask, wh, s, dt, maxd)


def kernel(features, adj_matrix, W1, a1, W2, a2, W3, a3):
    wh, s, dt, maxd = _mm(features, W1, a1)
    mask, wh, s, dt, maxd = _attn_mid(adj_matrix, wh, s, dt, maxd,
                                      W2, a2, first=True)
    wh, s, dt, maxd = _attn_mid(mask, wh, s, dt, maxd, W3, a3, first=False)
    return _attn_last(mask, wh, s, dt, maxd)


# speedup vs baseline: 1.0403x; 1.0403x over previous
import functools

import jax
import jax.numpy as jnp
from jax.experimental import pallas as pl
from jax.experimental.pallas import tpu as pltpu

ALPHA = 0.2
LOG2E = 1.4426950408889634
NHA = 640

BM = 512
BMM = 512


def _wh_outputs(i, wh, asd_ref, wh_ref, s_ref, dt_ref, maxd_ref):
    sd = jnp.dot(wh, asd_ref[...], preferred_element_type=jnp.float32)
    s_ref[...] = sd[:, 0:1]
    d = sd[:, 1:2]
    dt_ref[...] = d.T
    wh_ref[:, :512] = wh.astype(jnp.bfloat16)
    lane = jax.lax.broadcasted_iota(jnp.int32, (wh.shape[0], NHA - 512), 1)
    wh_ref[:, 512:] = jnp.where(lane == 0, 1.0, 0.0).astype(jnp.bfloat16)
    local_max = jnp.max(d, axis=0, keepdims=True)

    @pl.when(i == 0)
    def _first():
        maxd_ref[...] = local_max

    @pl.when(i > 0)
    def _rest():
        maxd_ref[...] = jnp.maximum(maxd_ref[...], local_max)


def _mm_kernel(h_ref, w_ref, asd_ref, wh_ref, s_ref, dt_ref, maxd_ref):
    i = pl.program_id(0)
    wh = jnp.dot(h_ref[...], w_ref[...], preferred_element_type=jnp.float32)
    _wh_outputs(i, wh, asd_ref, wh_ref, s_ref, dt_ref, maxd_ref)


def _wh_specs(n, bm):
    out_specs = [
        pl.BlockSpec((bm, NHA), lambda i: (i, 0)),
        pl.BlockSpec((bm, 1), lambda i: (i, 0)),
        pl.BlockSpec((1, bm), lambda i: (0, i)),
        pl.BlockSpec((1, 1), lambda i: (0, 0)),
    ]
    out_shape = [
        jax.ShapeDtypeStruct((n, NHA), jnp.bfloat16),
        jax.ShapeDtypeStruct((n, 1), jnp.float32),
        jax.ShapeDtypeStruct((1, n), jnp.float32),
        jax.ShapeDtypeStruct((1, 1), jnp.float32),
    ]
    return out_specs, out_shape


def _asd(a):
    nh = a.shape[0] // 2
    return jnp.concatenate([a[:nh], a[nh:]], axis=1) * LOG2E


def _mm(h, w, a):
    n, nin = h.shape
    out_specs, out_shape = _wh_specs(n, BMM)
    return pl.pallas_call(
        _mm_kernel,
        grid=(n // BMM,),
        in_specs=[
            pl.BlockSpec((BMM, nin), lambda i: (i, 0)),
            pl.BlockSpec((nin, w.shape[1]), lambda i: (0, 0)),
            pl.BlockSpec((w.shape[1], 2), lambda i: (0, 0)),
        ],
        out_specs=out_specs,
        out_shape=out_shape,
        compiler_params=pltpu.CompilerParams(
            dimension_semantics=("arbitrary",),
        ),
    )(h, w, _asd(a))


KC = 1024
KC3 = 2048


def _p_chunks(mask_chunk, s_ref, dt_ref, maxd_ref, kc):
    s = s_ref[...]
    x = s + maxd_ref[...]
    m = jnp.maximum(x, ALPHA * x)
    a = s - m
    b = ALPHA * s - m
    dt = dt_ref[...]
    out = []
    for c in range(dt.shape[1] // kc):
        dtc = dt[:, c * kc:(c + 1) * kc]
        t = jnp.maximum(a + dtc, b + ALPHA * dtc)
        out.append(jnp.exp2(t).astype(jnp.bfloat16) * mask_chunk(c))
    return out


def _pv(p16s, wh_ref, lo, hi, kc):
    pv = None
    for c, p16 in enumerate(p16s):
        part = jnp.dot(p16, wh_ref[pl.ds(c * kc, kc), lo:hi],
                       preferred_element_type=jnp.float32)
        pv = part if pv is None else pv + part
    return pv


def _attn_out(mask_chunk, wh_ref, s_ref, dt_ref, maxd_ref, kc):
    p16s = _p_chunks(mask_chunk, s_ref, dt_ref, maxd_ref, kc)
    pv = _pv(p16s, wh_ref, 0, NHA, kc)
    return pv[:, :512] / pv[:, 512:513]


def _attn_out_next(mask_chunk, wh_ref, s_ref, dt_ref, maxd_ref, wn_ref):
    h = _attn_out(mask_chunk, wh_ref, s_ref, dt_ref, maxd_ref, KC)
    return jnp.dot(jnp.maximum(h, 0.0), wn_ref[...],
                   preferred_element_type=jnp.float32)


def _attn1_kernel(adj_ref, wh_ref, s_ref, dt_ref, maxd_ref, wn_ref, asd_ref,
                  mask_ref, who_ref, so_ref, dto_ref, maxdo_ref):
    i = pl.program_id(0)
    masked = adj_ref[...] > 0.99
    mask_ref[...] = masked.astype(jnp.int8)

    def mask_chunk(c):
        return masked[:, c * KC:(c + 1) * KC].astype(jnp.bfloat16)

    wh = _attn_out_next(mask_chunk, wh_ref, s_ref, dt_ref, maxd_ref, wn_ref)
    _wh_outputs(i, wh, asd_ref, who_ref, so_ref, dto_ref, maxdo_ref)


def _attn2_kernel(mask_ref, wh_ref, s_ref, dt_ref, maxd_ref, wn_ref, asd_ref,
                  who_ref, so_ref, dto_ref, maxdo_ref):
    i = pl.program_id(0)
    def mask_chunk(c):
        return mask_ref[:, pl.ds(c * KC, KC)].astype(jnp.bfloat16)

    wh = _attn_out_next(mask_chunk, wh_ref, s_ref, dt_ref, maxd_ref, wn_ref)
    _wh_outputs(i, wh, asd_ref, who_ref, so_ref, dto_ref, maxdo_ref)


def _attn3_kernel(mask_ref, wh_ref, s_ref, dt_ref, maxd_ref, o_ref):
    def mask_chunk(c):
        return mask_ref[:, pl.ds(c * KC3, KC3)].astype(jnp.bfloat16)

    o_ref[...] = _attn_out(mask_chunk, wh_ref, s_ref, dt_ref, maxd_ref, KC3)


def _attn_specs(n, first):
    mat_dtype = jnp.float32 if first else jnp.bfloat16
    in_specs = [
        pl.BlockSpec((BM, n), lambda i: (i, 0)),
        pl.BlockSpec((n, NHA), lambda i: (0, 0)),
        pl.BlockSpec((BM, 1), lambda i: (i, 0)),
        pl.BlockSpec((1, n), lambda i: (0, 0)),
        pl.BlockSpec((1, 1), lambda i: (0, 0)),
    ]
    return mat_dtype, in_specs


def _attn_mid(mat, wh, s, dt, maxd, w_next, a_next, first):
    n = s.shape[0]
    _, in_specs = _attn_specs(n, first)
    in_specs += [
        pl.BlockSpec((512, 512), lambda i: (0, 0)),
        pl.BlockSpec((512, 2), lambda i: (0, 0)),
    ]
    out_specs, out_shape = _wh_specs(n, BM)
    if first:
        body = _attn1_kernel
        out_specs = [pl.BlockSpec((BM, n), lambda i: (i, 0))] + out_specs
        out_shape = [jax.ShapeDtypeStruct((n, n), jnp.int8)] + out_shape
    else:
        body = _attn2_kernel
    return pl.pallas_call(
        body,
        grid=(n // BM,),
        in_specs=in_specs,
        out_specs=out_specs,
        out_shape=out_shape,
        compiler_params=pltpu.CompilerParams(
            dimension_semantics=("arbitrary",),
        ),
    )(mat, wh, s, dt, maxd, w_next, _asd(a_next))


def _attn_last(mask, wh, s, dt, maxd):
    n = s.shape[0]
    _, in_specs = _attn_specs(n, False)
    return pl.pallas_call(
        _attn3_kernel,
        grid=(n // BM,),
        in_specs=in_specs,
        out_specs=pl.BlockSpec((BM, 512), lambda i: (i, 0)),
        out_shape=jax.ShapeDtypeStruct((n, 512), jnp.float32),
        compiler_params=pltpu.CompilerParams(
            dimension_semantics=("parallel",),
        ),
    )(mask, wh, s, dt, maxd)


def kernel(features, adj_matrix, W1, a1, W2, a2, W3, a3):
    wh, s, dt, maxd = _mm(features, W1, a1)
    mask, wh, s, dt, maxd = _attn_mid(adj_matrix, wh, s, dt, maxd,
                                      W2, a2, first=True)
    wh, s, dt, maxd = _attn_mid(mask, wh, s, dt, maxd, W3, a3, first=False)
    return _attn_last(mask, wh, s, dt, maxd)
